# baseline (speedup 1.0000x reference)
import functools

import jax
import jax.numpy as jnp
from jax import lax
from jax.experimental import pallas as pl
from jax.experimental.pallas import tpu as pltpu
from jax.experimental.pallas import tpu_sc as plsc

B = 4096
EMB = 32
PACK = 128 // EMB
VROWS = 25000

_NC, _NS = 2, 16
_NW = _NC * _NS
_BPW = B // _NW


def _gather_body(uidx_hbm, sidx_hbm, ut_hbm, st_hbm, uout_hbm, sout_hbm,
                 uidx_v, urows_v, sidx_v, srows_v, usem, ssem):
    wid = lax.axis_index("s") * _NC + lax.axis_index("c")
    base = wid * _BPW
    pltpu.sync_copy(uidx_hbm.at[pl.ds(base, _BPW)], uidx_v)
    pltpu.sync_copy(sidx_hbm.at[pl.ds(base, _BPW)], sidx_v)
    ucopy = pltpu.async_copy(ut_hbm.at[uidx_v], urows_v, usem)
    scopy = pltpu.async_copy(st_hbm.at[sidx_v], srows_v, ssem)
    ucopy.wait()
    scopy.wait()
    pltpu.sync_copy(urows_v, uout_hbm.at[pl.ds(base, _BPW)])
    pltpu.sync_copy(srows_v, sout_hbm.at[pl.ds(base, _BPW)])


@functools.cache
def _sc_gather():
    return pl.kernel(
        _gather_body,
        mesh=plsc.VectorSubcoreMesh(core_axis_name="c", subcore_axis_name="s",
                                    num_cores=_NC, num_subcores=_NS),
        out_type=[
            jax.ShapeDtypeStruct((B, 128), jnp.float32),
            jax.ShapeDtypeStruct((B, 128), jnp.float32),
        ],
        scratch_types=[
            pltpu.VMEM((_BPW,), jnp.int32),
            pltpu.VMEM((_BPW, 128), jnp.float32),
            pltpu.VMEM((_BPW,), jnp.int32),
            pltpu.VMEM((_BPW, 128), jnp.float32),
            pltpu.SemaphoreType.DMA,
            pltpu.SemaphoreType.DMA,
        ],
        compiler_params=pltpu.CompilerParams(use_tc_tiling_on_sc=True),
    )


def _mlp_body(uq_ref, sq_ref, u_ref, s_ref, w1u_ref, w1s_ref, b1_ref,
              w2_ref, b2_ref, w3_ref, b3_ref, out_ref):
    rows = u_ref.shape[0]
    lane_q = lax.broadcasted_iota(jnp.int32, (rows, 128), 1) >> 5
    um = jnp.where(lane_q == uq_ref[...], u_ref[...], 0.0)
    sm = jnp.where(lane_q == sq_ref[...], s_ref[...], 0.0)
    h = jnp.dot(um, w1u_ref[...], preferred_element_type=jnp.float32)
    h += jnp.dot(sm, w1s_ref[...], preferred_element_type=jnp.float32)
    h = jnp.maximum(h + b1_ref[...], 0.0)
    h = jnp.maximum(
        jnp.dot(h, w2_ref[...], preferred_element_type=jnp.float32)
        + b2_ref[...], 0.0)
    out_ref[...] = (
        jnp.dot(h, w3_ref[...], preferred_element_type=jnp.float32)
        + b3_ref[...])


def _mlp(uq, sq, u128, s128, W1u_rep, W1s_rep, b1, W2, b2, W3, b3):
    nb = 4
    rows = B // nb
    return pl.pallas_call(
        _mlp_body,
        grid=(nb,),
        in_specs=[
            pl.BlockSpec((rows, 1), lambda i: (i, 0)),
            pl.BlockSpec((rows, 1), lambda i: (i, 0)),
            pl.BlockSpec((rows, 128), lambda i: (i, 0)),
            pl.BlockSpec((rows, 128), lambda i: (i, 0)),
            pl.BlockSpec((128, 256), lambda i: (0, 0)),
            pl.BlockSpec((128, 256), lambda i: (0, 0)),
            pl.BlockSpec((1, 256), lambda i: (0, 0)),
            pl.BlockSpec((256, 64), lambda i: (0, 0)),
            pl.BlockSpec((1, 64), lambda i: (0, 0)),
            pl.BlockSpec((64, 1), lambda i: (0, 0)),
            pl.BlockSpec((1, 1), lambda i: (0, 0)),
        ],
        out_specs=pl.BlockSpec((rows, 1), lambda i: (i, 0)),
        out_shape=jax.ShapeDtypeStruct((B, 1), jnp.float32),
    )(uq, sq, u128, s128, W1u_rep, W1s_rep, b1, W2, b2, W3, b3)


@jax.jit
def kernel(user_id, song_id, user_table, song_table, W1, b1, W2, b2, W3, b3):
    uid = user_id.astype(jnp.int32)
    sid = song_id.astype(jnp.int32)
    ut128 = user_table[:PACK * VROWS].reshape(VROWS, 128)
    st128 = song_table[:PACK * VROWS].reshape(VROWS, 128)
    u128, s128 = _sc_gather()(uid >> 2, sid >> 2, ut128, st128)
    W1u_rep = jnp.tile(W1[:EMB], (PACK, 1))
    W1s_rep = jnp.tile(W1[EMB:], (PACK, 1))
    return _mlp((uid & 3).reshape(B, 1), (sid & 3).reshape(B, 1),
                u128, s128, W1u_rep, W1s_rep, b1.reshape(1, 256),
                W2, b2.reshape(1, 64), W3, b3.reshape(1, 1))


# speedup vs baseline: 1.6554x; 1.6554x over previous
import functools

import jax
import jax.numpy as jnp
from jax import lax
from jax.experimental import pallas as pl
from jax.experimental.pallas import tpu as pltpu
from jax.experimental.pallas import tpu_sc as plsc

B = 4096
EMB = 32
PITCH = 100001

_NC, _NS = 2, 16
_NW = _NC * _NS
_BPW = B // _NW


def _gather_body(uid_hbm, sid_hbm, ut_hbm, st_hbm, uout_hbm, sout_hbm,
                 ids_v, uidx_v, sidx_v, urows_v, srows_v, usem, ssem):
    wid = lax.axis_index("s") * _NC + lax.axis_index("c")
    base = wid * _BPW
    nv = _BPW // 16

    pltpu.sync_copy(uid_hbm.at[pl.ds(base, _BPW)], ids_v)
    for c in range(EMB):
        for k in range(nv):
            uidx_v[c, pl.ds(16 * k, 16)] = ids_v[pl.ds(16 * k, 16)] + c * PITCH
    pltpu.sync_copy(sid_hbm.at[pl.ds(base, _BPW)], ids_v)
    for c in range(EMB):
        for k in range(nv):
            sidx_v[c, pl.ds(16 * k, 16)] = ids_v[pl.ds(16 * k, 16)] + c * PITCH

    ucopies = [pltpu.async_copy(ut_hbm.at[uidx_v.at[c]], urows_v.at[c], usem)
               for c in range(EMB)]
    scopies = [pltpu.async_copy(st_hbm.at[sidx_v.at[c]], srows_v.at[c], ssem)
               for c in range(EMB)]
    for cp in ucopies:
        cp.wait()
    for cp in scopies:
        cp.wait()
    pltpu.sync_copy(urows_v, uout_hbm.at[:, pl.ds(base, _BPW)])
    pltpu.sync_copy(srows_v, sout_hbm.at[:, pl.ds(base, _BPW)])


@functools.cache
def _sc_gather():
    return pl.kernel(
        _gather_body,
        mesh=plsc.VectorSubcoreMesh(core_axis_name="c", subcore_axis_name="s",
                                    num_cores=_NC, num_subcores=_NS),
        out_type=[
            jax.ShapeDtypeStruct((EMB, B), jnp.float32),
            jax.ShapeDtypeStruct((EMB, B), jnp.float32),
        ],
        scratch_types=[
            pltpu.VMEM((_BPW,), jnp.int32),
            pltpu.VMEM((EMB, _BPW), jnp.int32),
            pltpu.VMEM((EMB, _BPW), jnp.int32),
            pltpu.VMEM((EMB, _BPW), jnp.float32),
            pltpu.VMEM((EMB, _BPW), jnp.float32),
            pltpu.SemaphoreType.DMA,
            pltpu.SemaphoreType.DMA,
        ],
        compiler_params=pltpu.CompilerParams(use_tc_tiling_on_sc=False),
    )


def _mlp_body(u_ref, s_ref, w1u_ref, w1s_ref, b1_ref, w2_ref, b2_ref,
              w3_ref, b3_ref, out_ref):
    cdim = (((0,), (0,)), ((), ()))
    h = lax.dot_general(u_ref[...], w1u_ref[...], cdim,
                        preferred_element_type=jnp.float32)
    h += lax.dot_general(s_ref[...], w1s_ref[...], cdim,
                         preferred_element_type=jnp.float32)
    h = jnp.maximum(h + b1_ref[...], 0.0)
    h = jnp.maximum(
        jnp.dot(h, w2_ref[...], preferred_element_type=jnp.float32)
        + b2_ref[...], 0.0)
    out_ref[...] = (
        jnp.dot(h, w3_ref[...], preferred_element_type=jnp.float32)
        + b3_ref[...])


def _mlp(u, s, W1u, W1s, b1, W2, b2, W3, b3):
    nb = 4
    cols = B // nb
    return pl.pallas_call(
        _mlp_body,
        grid=(nb,),
        in_specs=[
            pl.BlockSpec((EMB, cols), lambda i: (0, i)),
            pl.BlockSpec((EMB, cols), lambda i: (0, i)),
            pl.BlockSpec((EMB, 256), lambda i: (0, 0)),
            pl.BlockSpec((EMB, 256), lambda i: (0, 0)),
            pl.BlockSpec((1, 256), lambda i: (0, 0)),
            pl.BlockSpec((256, 64), lambda i: (0, 0)),
            pl.BlockSpec((1, 64), lambda i: (0, 0)),
            pl.BlockSpec((64, 1), lambda i: (0, 0)),
            pl.BlockSpec((1, 1), lambda i: (0, 0)),
        ],
        out_specs=pl.BlockSpec((cols, 1), lambda i: (i, 0)),
        out_shape=jax.ShapeDtypeStruct((B, 1), jnp.float32),
    )(u, s, W1u, W1s, b1, W2, b2, W3, b3)


@jax.jit
def kernel(user_id, song_id, user_table, song_table, W1, b1, W2, b2, W3, b3):
    uid = user_id.astype(jnp.int32)
    sid = song_id.astype(jnp.int32)
    ulin = user_table.T.reshape(-1)
    slin = song_table.T.reshape(-1)
    u_emb, s_emb = _sc_gather()(uid, sid, ulin, slin)
    return _mlp(u_emb, s_emb, W1[:EMB], W1[EMB:], b1.reshape(1, 256),
                W2, b2.reshape(1, 64), W3, b3.reshape(1, 1))


# speedup vs baseline: 1.9379x; 1.1707x over previous
import functools

import jax
import jax.numpy as jnp
from jax import lax
from jax.experimental import pallas as pl
from jax.experimental.pallas import tpu as pltpu
from jax.experimental.pallas import tpu_sc as plsc

B = 4096
EMB = 32
PITCH = 100001

_NC, _NS = 2, 16
_NW = _NC * _NS
_BPW = B // _NW


def _gather_body(id_hbm, t_hbm, out_hbm, ids_v, idx_v, rows_v, sem):
    wid = lax.axis_index("s") * _NC + lax.axis_index("c")
    base = wid * _BPW
    nv = _BPW // 16

    pltpu.sync_copy(id_hbm.at[pl.ds(base, _BPW)], ids_v)
    for c in range(EMB):
        for k in range(nv):
            idx_v[c, pl.ds(16 * k, 16)] = ids_v[pl.ds(16 * k, 16)] + c * PITCH
    copies = [pltpu.async_copy(t_hbm.at[idx_v.at[c]], rows_v.at[c], sem)
              for c in range(EMB)]
    for cp in copies:
        cp.wait()
    pltpu.sync_copy(rows_v, out_hbm.at[:, pl.ds(base, _BPW)])


@functools.cache
def _sc_gather():
    return pl.kernel(
        _gather_body,
        mesh=plsc.VectorSubcoreMesh(core_axis_name="c", subcore_axis_name="s",
                                    num_cores=_NC, num_subcores=_NS),
        out_type=jax.ShapeDtypeStruct((EMB, B), jnp.float32),
        scratch_types=[
            pltpu.VMEM((_BPW,), jnp.int32),
            pltpu.VMEM((EMB, _BPW), jnp.int32),
            pltpu.VMEM((EMB, _BPW), jnp.float32),
            pltpu.SemaphoreType.DMA,
        ],
        compiler_params=pltpu.CompilerParams(use_tc_tiling_on_sc=False),
    )


def _mlp_body(u_ref, s_ref, w1u_ref, w1s_ref, b1_ref, w2_ref, b2_ref,
              w3_ref, b3_ref, out_ref):
    cdim = (((0,), (0,)), ((), ()))
    h = lax.dot_general(w1u_ref[...], u_ref[...], cdim,
                        preferred_element_type=jnp.float32)
    h += lax.dot_general(w1s_ref[...], s_ref[...], cdim,
                         preferred_element_type=jnp.float32)
    h = jnp.maximum(h + b1_ref[...], 0.0)
    h = jnp.maximum(
        lax.dot_general(w2_ref[...], h, cdim,
                        preferred_element_type=jnp.float32) + b2_ref[...], 0.0)
    out_ref[...] = (
        lax.dot_general(w3_ref[...], h, cdim,
                        preferred_element_type=jnp.float32) + b3_ref[...])


def _mlp(u, s, W1u, W1s, b1, W2, b2, W3, b3):
    nb = 4
    cols = B // nb
    return pl.pallas_call(
        _mlp_body,
        grid=(nb,),
        in_specs=[
            pl.BlockSpec((EMB, cols), lambda i: (0, i)),
            pl.BlockSpec((EMB, cols), lambda i: (0, i)),
            pl.BlockSpec((EMB, 256), lambda i: (0, 0)),
            pl.BlockSpec((EMB, 256), lambda i: (0, 0)),
            pl.BlockSpec((256, 1), lambda i: (0, 0)),
            pl.BlockSpec((256, 64), lambda i: (0, 0)),
            pl.BlockSpec((64, 1), lambda i: (0, 0)),
            pl.BlockSpec((64, 1), lambda i: (0, 0)),
            pl.BlockSpec((1, 1), lambda i: (0, 0)),
        ],
        out_specs=pl.BlockSpec((1, cols), lambda i: (0, i)),
        out_shape=jax.ShapeDtypeStruct((1, B), jnp.float32),
    )(u, s, W1u, W1s, b1, W2, b2, W3, b3)


@jax.jit
def kernel(user_id, song_id, user_table, song_table, W1, b1, W2, b2, W3, b3):
    uid = user_id.astype(jnp.int32)
    sid = song_id.astype(jnp.int32)
    g = _sc_gather()
    ulin = user_table.T.reshape(-1)
    u_emb = g(uid, ulin)
    slin = song_table.T.reshape(-1)
    s_emb = g(sid, slin)
    out_t = _mlp(u_emb, s_emb, W1[:EMB], W1[EMB:], b1.reshape(256, 1),
                 W2, b2.reshape(64, 1), W3, b3.reshape(1, 1))
    return out_t.T


# speedup vs baseline: 1.9385x; 1.0003x over previous
import functools

import jax
import jax.numpy as jnp
from jax import lax
from jax.experimental import pallas as pl
from jax.experimental.pallas import tpu as pltpu
from jax.experimental.pallas import tpu_sc as plsc

B = 4096
EMB = 32
PITCH = 100001

_NC, _NS = 2, 16
_NW = _NC * _NS
_BPW = B // _NW


def _gather_body(id_hbm, t_hbm, out_hbm, ids_v, idx_v, rows_v, sem):
    wid = lax.axis_index("s") * _NC + lax.axis_index("c")
    base = wid * _BPW
    nv = _BPW // 16

    pltpu.sync_copy(id_hbm.at[pl.ds(base, _BPW)], ids_v)
    copies = []
    for c in range(EMB):
        for k in range(nv):
            idx_v[c, pl.ds(16 * k, 16)] = ids_v[pl.ds(16 * k, 16)] + c * PITCH
        copies.append(pltpu.async_copy(t_hbm.at[idx_v.at[c]], rows_v.at[c],
                                       sem))
    for cp in copies:
        cp.wait()
    pltpu.sync_copy(rows_v, out_hbm.at[:, pl.ds(base, _BPW)])


@functools.cache
def _sc_gather():
    return pl.kernel(
        _gather_body,
        mesh=plsc.VectorSubcoreMesh(core_axis_name="c", subcore_axis_name="s",
                                    num_cores=_NC, num_subcores=_NS),
        out_type=jax.ShapeDtypeStruct((EMB, B), jnp.float32),
        scratch_types=[
            pltpu.VMEM((_BPW,), jnp.int32),
            pltpu.VMEM((EMB, _BPW), jnp.int32),
            pltpu.VMEM((EMB, _BPW), jnp.float32),
            pltpu.SemaphoreType.DMA,
        ],
        compiler_params=pltpu.CompilerParams(use_tc_tiling_on_sc=False),
    )


def _mlp_body(u_ref, s_ref, w1u_ref, w1s_ref, b1_ref, w2_ref, b2_ref,
              w3_ref, b3_ref, out_ref):
    cdim = (((0,), (0,)), ((), ()))
    h = lax.dot_general(w1u_ref[...], u_ref[...], cdim,
                        preferred_element_type=jnp.float32)
    h += lax.dot_general(w1s_ref[...], s_ref[...], cdim,
                         preferred_element_type=jnp.float32)
    h = jnp.maximum(h + b1_ref[...], 0.0)
    h = jnp.maximum(
        lax.dot_general(w2_ref[...], h, cdim,
                        preferred_element_type=jnp.float32) + b2_ref[...], 0.0)
    out_ref[...] = (
        lax.dot_general(w3_ref[...], h, cdim,
                        preferred_element_type=jnp.float32) + b3_ref[...])


def _mlp(u, s, W1u, W1s, b1, W2, b2, W3, b3):
    nb = 4
    cols = B // nb
    return pl.pallas_call(
        _mlp_body,
        grid=(nb,),
        in_specs=[
            pl.BlockSpec((EMB, cols), lambda i: (0, i)),
            pl.BlockSpec((EMB, cols), lambda i: (0, i)),
            pl.BlockSpec((EMB, 256), lambda i: (0, 0)),
            pl.BlockSpec((EMB, 256), lambda i: (0, 0)),
            pl.BlockSpec((256, 1), lambda i: (0, 0)),
            pl.BlockSpec((256, 64), lambda i: (0, 0)),
            pl.BlockSpec((64, 1), lambda i: (0, 0)),
            pl.BlockSpec((64, 1), lambda i: (0, 0)),
            pl.BlockSpec((1, 1), lambda i: (0, 0)),
        ],
        out_specs=pl.BlockSpec((1, cols), lambda i: (0, i)),
        out_shape=jax.ShapeDtypeStruct((1, B), jnp.float32),
    )(u, s, W1u, W1s, b1, W2, b2, W3, b3)


@jax.jit
def kernel(user_id, song_id, user_table, song_table, W1, b1, W2, b2, W3, b3):
    uid = user_id.astype(jnp.int32)
    sid = song_id.astype(jnp.int32)
    g = _sc_gather()
    ulin = user_table.T.reshape(-1)
    u_emb = g(uid, ulin)
    slin = song_table.T.reshape(-1)
    s_emb = g(sid, slin)
    out_t = _mlp(u_emb, s_emb, W1[:EMB], W1[EMB:], b1.reshape(256, 1),
                 W2, b2.reshape(64, 1), W3, b3.reshape(1, 1))
    return out_t.T


# speedup vs baseline: 1.9763x; 1.0195x over previous
import functools

import jax
import jax.numpy as jnp
from jax import lax
from jax.experimental import pallas as pl
from jax.experimental.pallas import tpu as pltpu
from jax.experimental.pallas import tpu_sc as plsc

B = 4096
EMB = 32
PITCH = 100001

_NC, _NS = 2, 16
_NW = _NC * _NS
_BPW = B // _NW


def _gather_body(id_hbm, t_hbm, out_hbm, ids_v, idx_v, rows_v, sem):
    wid = lax.axis_index("s") * _NC + lax.axis_index("c")
    base = wid * _BPW
    nv = _BPW // 16

    pltpu.sync_copy(id_hbm.at[pl.ds(base, _BPW)], ids_v)
    copies = []
    for c in range(EMB):
        for k in range(nv):
            idx_v[c, pl.ds(16 * k, 16)] = ids_v[pl.ds(16 * k, 16)] + c * PITCH
        copies.append(pltpu.async_copy(t_hbm.at[idx_v.at[c]], rows_v.at[c],
                                       sem))
    for cp in copies:
        cp.wait()
    pltpu.sync_copy(rows_v, out_hbm.at[:, pl.ds(base, _BPW)])


@functools.cache
def _sc_gather():
    return pl.kernel(
        _gather_body,
        mesh=plsc.VectorSubcoreMesh(core_axis_name="c", subcore_axis_name="s",
                                    num_cores=_NC, num_subcores=_NS),
        out_type=jax.ShapeDtypeStruct((EMB, B), jnp.float32),
        scratch_types=[
            pltpu.VMEM((_BPW,), jnp.int32),
            pltpu.VMEM((EMB, _BPW), jnp.int32),
            pltpu.VMEM((EMB, _BPW), jnp.float32),
            pltpu.SemaphoreType.DMA,
        ],
        compiler_params=pltpu.CompilerParams(use_tc_tiling_on_sc=False),
    )


def _mlp_body(u_ref, s_ref, w1u_ref, w1s_ref, b1_ref, w2_ref, b2_ref,
              w3_ref, b3_ref, out_ref):
    cdim = (((0,), (0,)), ((), ()))
    h = lax.dot_general(w1u_ref[...], u_ref[...], cdim,
                        preferred_element_type=jnp.float32)
    h += lax.dot_general(w1s_ref[...], s_ref[...], cdim,
                         preferred_element_type=jnp.float32)
    h = jnp.maximum(h + b1_ref[...], 0.0)
    h = jnp.maximum(
        lax.dot_general(w2_ref[...], h, cdim,
                        preferred_element_type=jnp.float32) + b2_ref[...], 0.0)
    out_ref[...] = (
        lax.dot_general(w3_ref[...], h, cdim,
                        preferred_element_type=jnp.float32) + b3_ref[...])


def _mlp(u, s, W1u, W1s, b1, W2, b2, W3, b3):
    nb = 1
    cols = B // nb
    return pl.pallas_call(
        _mlp_body,
        grid=(nb,),
        in_specs=[
            pl.BlockSpec((EMB, cols), lambda i: (0, i)),
            pl.BlockSpec((EMB, cols), lambda i: (0, i)),
            pl.BlockSpec((EMB, 256), lambda i: (0, 0)),
            pl.BlockSpec((EMB, 256), lambda i: (0, 0)),
            pl.BlockSpec((256, 1), lambda i: (0, 0)),
            pl.BlockSpec((256, 64), lambda i: (0, 0)),
            pl.BlockSpec((64, 1), lambda i: (0, 0)),
            pl.BlockSpec((64, 1), lambda i: (0, 0)),
            pl.BlockSpec((1, 1), lambda i: (0, 0)),
        ],
        out_specs=pl.BlockSpec((1, cols), lambda i: (0, i)),
        out_shape=jax.ShapeDtypeStruct((1, B), jnp.float32),
    )(u, s, W1u, W1s, b1, W2, b2, W3, b3)


@jax.jit
def kernel(user_id, song_id, user_table, song_table, W1, b1, W2, b2, W3, b3):
    uid = user_id.astype(jnp.int32)
    sid = song_id.astype(jnp.int32)
    g = _sc_gather()
    ulin = user_table.T.reshape(-1)
    u_emb = g(uid, ulin)
    slin = song_table.T.reshape(-1)
    s_emb = g(sid, slin)
    out_t = _mlp(u_emb, s_emb, W1[:EMB], W1[EMB:], b1.reshape(256, 1),
                 W2, b2.reshape(64, 1), W3, b3.reshape(1, 1))
    return out_t.T
